# Initial kernel scaffold; baseline (speedup 1.0000x reference)
import jax
import jax.numpy as jnp
from jax.experimental import pallas as pl


def kernel(user_time_W, user_cat_W, POI_time_W, POI_cat_W, time_tr_W, time_proj_W, cat_tr_W, cat_proj_W, pos_u, pos_t, pos_p, pos_c, neg_u, neg_t, neg_p, neg_u2, neg_c, neg_p2, NS):
    raise NotImplementedError("write your pallas kernel here")


# speedup vs baseline: 2.2475x; 2.2475x over previous
import functools

import jax
import jax.numpy as jnp
from jax import lax
from jax.experimental import pallas as pl
from jax.experimental.pallas import tpu as pltpu
from jax.experimental.pallas import tpu_sc as plsc

E = 64
BK = 256
B = 4096
K_TIME = 168
K_CAT = 400
CHUNK = 128
NBUF = 4


def _sc_gather2(user_both, poi_both, idx_user, idx_poi):
    n_rows = idx_user.shape[0]
    d = user_both.shape[1]
    info = plsc.get_sparse_core_info()
    nw = info.num_cores * info.num_subcores
    bpw = n_rows // nw
    nchunk = bpw // CHUNK
    mesh = plsc.VectorSubcoreMesh(core_axis_name="c", subcore_axis_name="s")
    out_type = [jax.ShapeDtypeStruct((n_rows, d), jnp.float32)] * 2
    scratch = [pltpu.VMEM((CHUNK,), jnp.int32) for _ in range(NBUF)]
    scratch += [pltpu.VMEM((NBUF, CHUNK, d), jnp.float32),
                pltpu.SemaphoreType.DMA]

    @functools.partial(pl.kernel, mesh=mesh, out_type=out_type,
                       scratch_types=scratch)
    def gk(ub, pb, iu, ip, ou, op, *sc):
        idxv = sc[:NBUF]
        rows = sc[NBUF]
        sem = sc[NBUF + 1]
        wid = lax.axis_index("s") * info.num_cores + lax.axis_index("c")
        base = wid * bpw
        tasks = ([(ub, iu, ou, k) for k in range(nchunk)]
                 + [(pb, ip, op, k) for k in range(nchunk)])
        descs = [None] * len(tasks)

        def writeout(k):
            _, _, out, kk = tasks[k]
            pltpu.sync_copy(rows.at[k % NBUF],
                            out.at[pl.ds(base + kk * CHUNK, CHUNK)])

        for k, (tab, idx, _, kk) in enumerate(tasks):
            if k >= NBUF:
                descs[k - NBUF].wait()
                writeout(k - NBUF)
            pltpu.sync_copy(idx.at[pl.ds(base + kk * CHUNK, CHUNK)],
                            idxv[k % NBUF])
            descs[k] = pltpu.async_copy(tab.at[idxv[k % NBUF]],
                                        rows.at[k % NBUF], sem)
        for k in range(len(tasks) - NBUF, len(tasks)):
            descs[k].wait()
            writeout(k)

    return gk(user_both, poi_both, idx_user, idx_poi)


def _log_sigmoid(x):
    return -(jnp.maximum(-x, 0.0) + jnp.log(1.0 + jnp.exp(-jnp.abs(x))))


def _tc_score(tcol, ccol, gu, gp, WtT, WcT, trtT, trcT, rep, tile):
    grid = (6 * B // BK,)
    bps = B // BK

    def body(t_ref, c_ref, gua_ref, gub_ref, gpa_ref, gpb_ref,
             wtT_ref, wcT_ref, trtT_ref, trcT_ref, rep_ref, tile_ref,
             pos_ref, neg_ref):
        j = pl.program_id(0)

        def score(u, p, wT, trT, idx, K):
            prep = jnp.dot(p.astype(jnp.bfloat16), rep_ref[...],
                           preferred_element_type=jnp.float32)
            util = jnp.dot(u.astype(jnp.bfloat16), tile_ref[...],
                           preferred_element_type=jnp.float32)
            v = (prep * util).astype(jnp.bfloat16)
            y = jnp.dot(v, wT, preferred_element_type=jnp.float32)
            y = y + jnp.dot(p, trT, preferred_element_type=jnp.float32)
            oh = (lax.broadcasted_iota(jnp.int32, (BK, K), 1) == idx)
            return jnp.sum(y * oh.astype(jnp.float32), axis=1, keepdims=True)

        ut = gua_ref[:, :E]
        uc = gub_ref[:, E:]
        pt = gpa_ref[:, :E]
        pc = gpb_ref[:, E:]
        s_t = score(ut, pt, wtT_ref[...], trtT_ref[...], t_ref[...], K_TIME)
        s_c = score(uc, pc, wcT_ref[...], trcT_ref[...], c_ref[...], K_CAT)
        set_id = j // bps

        @pl.when(j == 0)
        def _():
            neg_ref[...] = jnp.zeros_like(neg_ref)

        @pl.when(set_id == 0)
        def _():
            pos_ref[...] = -(_log_sigmoid(s_t) + _log_sigmoid(s_c))

        @pl.when(set_id > 0)
        def _():
            part = jnp.sum(_log_sigmoid(-s_t) + _log_sigmoid(-s_c))
            rr = lax.broadcasted_iota(jnp.int32, (8, 128), 0)
            cc = lax.broadcasted_iota(jnp.int32, (8, 128), 1)
            m = (rr == (set_id - 1)) & (cc == 0)
            neg_ref[...] = neg_ref[...] + jnp.where(m, -part, 0.0)

    const = lambda j: (0, 0)
    row = lambda j: (j, 0)
    second = lambda j: (jnp.where(j < bps, j, j + 5 * bps), 0)
    return pl.pallas_call(
        body,
        grid=grid,
        in_specs=[
            pl.BlockSpec((BK, 1), row),
            pl.BlockSpec((BK, 1), row),
            pl.BlockSpec((BK, 2 * E), row),
            pl.BlockSpec((BK, 2 * E), second),
            pl.BlockSpec((BK, 2 * E), row),
            pl.BlockSpec((BK, 2 * E), second),
            pl.BlockSpec((E * E, K_TIME), const),
            pl.BlockSpec((E * E, K_CAT), const),
            pl.BlockSpec((E, K_TIME), const),
            pl.BlockSpec((E, K_CAT), const),
            pl.BlockSpec((E, E * E), const),
            pl.BlockSpec((E, E * E), const),
        ],
        out_specs=[
            pl.BlockSpec((BK, 1), lambda j: (jnp.minimum(j, bps - 1), 0)),
            pl.BlockSpec((8, 128), const),
        ],
        out_shape=[
            jax.ShapeDtypeStruct((B, 1), jnp.float32),
            jax.ShapeDtypeStruct((8, 128), jnp.float32),
        ],
    )(tcol, ccol, gu, gu, gp, gp, WtT, WcT, trtT, trcT, rep, tile)


def kernel(user_time_W, user_cat_W, POI_time_W, POI_cat_W, time_tr_W,
           time_proj_W, cat_tr_W, cat_proj_W, pos_u, pos_t, pos_p, pos_c,
           neg_u, neg_t, neg_p, neg_u2, neg_c, neg_p2, NS):
    i32 = jnp.int32
    idx_user = jnp.concatenate(
        [pos_u, neg_u.reshape(-1), neg_u2.reshape(-1)]).astype(i32)
    idx_poi = jnp.concatenate(
        [pos_p, neg_p.reshape(-1), neg_p2.reshape(-1)]).astype(i32)
    tcol = jnp.concatenate([pos_t, neg_t.reshape(-1)]).astype(i32).reshape(-1, 1)
    ccol = jnp.concatenate([pos_c, neg_c.reshape(-1)]).astype(i32).reshape(-1, 1)

    user_both = jnp.concatenate([user_time_W, user_cat_W], axis=1)
    poi_both = jnp.concatenate([POI_time_W, POI_cat_W], axis=1)
    gu, gp = _sc_gather2(user_both, poi_both, idx_user, idx_poi)

    WtT = time_proj_W.T.astype(jnp.bfloat16)
    WcT = cat_proj_W.T.astype(jnp.bfloat16)
    trtT = time_tr_W.T
    trcT = cat_tr_W.T
    jj = jnp.arange(E * E)
    rr = jnp.arange(E)
    rep = (jj[None, :] // E == rr[:, None]).astype(jnp.bfloat16)
    tile = (jj[None, :] % E == rr[:, None]).astype(jnp.bfloat16)

    pos2d, neg2d = _tc_score(tcol, ccol, gu, gp, WtT, WcT, trtT, trcT,
                             rep, tile)
    pos = pos2d.reshape(-1)
    neg = neg2d[:neg_u.shape[0], 0]
    return (pos, neg)


# speedup vs baseline: 2.3096x; 1.0276x over previous
import functools

import jax
import jax.numpy as jnp
from jax import lax
from jax.experimental import pallas as pl
from jax.experimental.pallas import tpu as pltpu
from jax.experimental.pallas import tpu_sc as plsc

E = 64
BK = 512
B = 4096
K_TIME = 168
K_CAT = 400
CHUNK = 128
NBUF = 4


def _sc_gather2(user_both, poi_both, idx_user, idx_poi):
    n_rows = idx_user.shape[0]
    d = user_both.shape[1]
    info = plsc.get_sparse_core_info()
    nw = info.num_cores * info.num_subcores
    bpw = n_rows // nw
    nchunk = bpw // CHUNK
    mesh = plsc.VectorSubcoreMesh(core_axis_name="c", subcore_axis_name="s")
    out_type = [jax.ShapeDtypeStruct((n_rows, d), jnp.float32)] * 2
    scratch = [pltpu.VMEM((CHUNK,), jnp.int32) for _ in range(NBUF)]
    scratch += [pltpu.VMEM((NBUF, CHUNK, d), jnp.float32),
                pltpu.SemaphoreType.DMA]

    @functools.partial(pl.kernel, mesh=mesh, out_type=out_type,
                       scratch_types=scratch)
    def gk(ub, pb, iu, ip, ou, op, *sc):
        idxv = sc[:NBUF]
        rows = sc[NBUF]
        sem = sc[NBUF + 1]
        wid = lax.axis_index("s") * info.num_cores + lax.axis_index("c")
        base = wid * bpw
        tasks = ([(ub, iu, ou, k) for k in range(nchunk)]
                 + [(pb, ip, op, k) for k in range(nchunk)])
        descs = [None] * len(tasks)

        def writeout(k):
            _, _, out, kk = tasks[k]
            pltpu.sync_copy(rows.at[k % NBUF],
                            out.at[pl.ds(base + kk * CHUNK, CHUNK)])

        for k, (tab, idx, _, kk) in enumerate(tasks):
            if k >= NBUF:
                descs[k - NBUF].wait()
                writeout(k - NBUF)
            pltpu.sync_copy(idx.at[pl.ds(base + kk * CHUNK, CHUNK)],
                            idxv[k % NBUF])
            descs[k] = pltpu.async_copy(tab.at[idxv[k % NBUF]],
                                        rows.at[k % NBUF], sem)
        for k in range(len(tasks) - NBUF, len(tasks)):
            descs[k].wait()
            writeout(k)

    return gk(user_both, poi_both, idx_user, idx_poi)


def _log_sigmoid(x):
    return -(jnp.maximum(-x, 0.0) + jnp.log(1.0 + jnp.exp(-jnp.abs(x))))


def _tc_score(tcol, ccol, gu, gp, WtT, WcT, trtT, trcT, rep, tile):
    grid = (6 * B // BK,)
    bps = B // BK

    def body(t_ref, c_ref, gua_ref, gub_ref, gpa_ref, gpb_ref,
             wtT_ref, wcT_ref, trtT_ref, trcT_ref, rep_ref, tile_ref,
             pos_ref, neg_ref):
        j = pl.program_id(0)

        def score(u, p, wT, trT, idx, K):
            prep = jnp.dot(p.astype(jnp.bfloat16), rep_ref[...],
                           preferred_element_type=jnp.float32)
            util = jnp.dot(u.astype(jnp.bfloat16), tile_ref[...],
                           preferred_element_type=jnp.float32)
            v = (prep * util).astype(jnp.bfloat16)
            y = jnp.dot(v, wT, preferred_element_type=jnp.float32)
            y = y + jnp.dot(p, trT, preferred_element_type=jnp.float32)
            oh = (lax.broadcasted_iota(jnp.int32, (BK, K), 1) == idx)
            return jnp.sum(y * oh.astype(jnp.float32), axis=1, keepdims=True)

        ut = gua_ref[:, :E]
        uc = gub_ref[:, E:]
        pt = gpa_ref[:, :E]
        pc = gpb_ref[:, E:]
        s_t = score(ut, pt, wtT_ref[...], trtT_ref[...], t_ref[...], K_TIME)
        s_c = score(uc, pc, wcT_ref[...], trcT_ref[...], c_ref[...], K_CAT)
        set_id = j // bps

        @pl.when(j == 0)
        def _():
            neg_ref[...] = jnp.zeros_like(neg_ref)

        @pl.when(set_id == 0)
        def _():
            pos_ref[...] = -(_log_sigmoid(s_t) + _log_sigmoid(s_c))

        @pl.when(set_id > 0)
        def _():
            part = jnp.sum(_log_sigmoid(-s_t) + _log_sigmoid(-s_c))
            rr = lax.broadcasted_iota(jnp.int32, (8, 128), 0)
            cc = lax.broadcasted_iota(jnp.int32, (8, 128), 1)
            m = (rr == (set_id - 1)) & (cc == 0)
            neg_ref[...] = neg_ref[...] + jnp.where(m, -part, 0.0)

    const = lambda j: (0, 0)
    row = lambda j: (j, 0)
    second = lambda j: (jnp.where(j < bps, j, j + 5 * bps), 0)
    return pl.pallas_call(
        body,
        grid=grid,
        in_specs=[
            pl.BlockSpec((BK, 1), row),
            pl.BlockSpec((BK, 1), row),
            pl.BlockSpec((BK, 2 * E), row),
            pl.BlockSpec((BK, 2 * E), second),
            pl.BlockSpec((BK, 2 * E), row),
            pl.BlockSpec((BK, 2 * E), second),
            pl.BlockSpec((E * E, K_TIME), const),
            pl.BlockSpec((E * E, K_CAT), const),
            pl.BlockSpec((E, K_TIME), const),
            pl.BlockSpec((E, K_CAT), const),
            pl.BlockSpec((E, E * E), const),
            pl.BlockSpec((E, E * E), const),
        ],
        out_specs=[
            pl.BlockSpec((BK, 1), lambda j: (jnp.minimum(j, bps - 1), 0)),
            pl.BlockSpec((8, 128), const),
        ],
        out_shape=[
            jax.ShapeDtypeStruct((B, 1), jnp.float32),
            jax.ShapeDtypeStruct((8, 128), jnp.float32),
        ],
    )(tcol, ccol, gu, gu, gp, gp, WtT, WcT, trtT, trcT, rep, tile)


def kernel(user_time_W, user_cat_W, POI_time_W, POI_cat_W, time_tr_W,
           time_proj_W, cat_tr_W, cat_proj_W, pos_u, pos_t, pos_p, pos_c,
           neg_u, neg_t, neg_p, neg_u2, neg_c, neg_p2, NS):
    i32 = jnp.int32
    idx_user = jnp.concatenate(
        [pos_u, neg_u.reshape(-1), neg_u2.reshape(-1)]).astype(i32)
    idx_poi = jnp.concatenate(
        [pos_p, neg_p.reshape(-1), neg_p2.reshape(-1)]).astype(i32)
    tcol = jnp.concatenate([pos_t, neg_t.reshape(-1)]).astype(i32).reshape(-1, 1)
    ccol = jnp.concatenate([pos_c, neg_c.reshape(-1)]).astype(i32).reshape(-1, 1)

    user_both = jnp.concatenate([user_time_W, user_cat_W], axis=1)
    poi_both = jnp.concatenate([POI_time_W, POI_cat_W], axis=1)
    gu, gp = _sc_gather2(user_both, poi_both, idx_user, idx_poi)

    WtT = time_proj_W.T.astype(jnp.bfloat16)
    WcT = cat_proj_W.T.astype(jnp.bfloat16)
    trtT = time_tr_W.T
    trcT = cat_tr_W.T
    jj = jnp.arange(E * E)
    rr = jnp.arange(E)
    rep = (jj[None, :] // E == rr[:, None]).astype(jnp.bfloat16)
    tile = (jj[None, :] % E == rr[:, None]).astype(jnp.bfloat16)

    pos2d, neg2d = _tc_score(tcol, ccol, gu, gp, WtT, WcT, trtT, trcT,
                             rep, tile)
    pos = pos2d.reshape(-1)
    neg = neg2d[:neg_u.shape[0], 0]
    return (pos, neg)
